# Initial kernel scaffold; baseline (speedup 1.0000x reference)
import jax
import jax.numpy as jnp
from jax.experimental import pallas as pl


def kernel(x, edge_index1, edge_index2, W1l, b1l, W1r, W2l, b2l, W2r):
    raise NotImplementedError("write your pallas kernel here")


# speedup vs baseline: 8.6897x; 8.6897x over previous
import functools

import jax
import jax.numpy as jnp
from jax import lax
from jax.experimental import pallas as pl
from jax.experimental.pallas import tpu as pltpu
from jax.experimental.pallas import tpu_sc as plsc

N0, N1, N2 = 10000, 5000, 2500
E1, E2 = 320000, 160000
D = 128
N1P, N2P = 5120, 2560
NC, NS = 2, 16
NW = NC * NS
CH = 128

F32 = jnp.float32


def _chunks_of(total, step):
    out, off = [], 0
    while off < total:
        n = min(step, total - off)
        out.append((off, n))
        off += n
    return out


def _make_sc_agg(n_table, E, Np):
    per_w = E // NW
    assert per_w * NW == E
    nch = per_w // CH
    tail = per_w - nch * CH
    sl = Np // NS
    assert sl * NS == Np and sl % 16 == 0

    @functools.partial(
        pl.kernel,
        out_type=(
            jax.ShapeDtypeStruct((NC, Np, D), F32),
            jax.ShapeDtypeStruct((NC * Np,), F32),
        ),
        mesh=plsc.VectorSubcoreMesh(core_axis_name="c", subcore_axis_name="s"),
        scratch_types=[
            pltpu.VMEM((CH,), jnp.int32),
            pltpu.VMEM((CH,), jnp.int32),
            pltpu.VMEM((max(tail, 8),), jnp.int32),
            pltpu.VMEM((max(tail, 8),), jnp.int32),
            pltpu.VMEM((CH, D), F32),
            pltpu.VMEM((CH,), F32),
            pltpu.VMEM((64, D), F32),
            pltpu.VMEM((Np,), F32),
            pltpu.VMEM_SHARED((Np, D), F32),
            pltpu.VMEM_SHARED((Np,), F32),
            pltpu.SemaphoreType.DMA,
        ],
    )
    def agg(table, src, dst, sum_out, cnt_out,
            src_idx, dst_idx, src_t, dst_t, rows, ones_r, zbuf, cbuf,
            acc, cnt, gsem):
        c = lax.axis_index("c")
        s = lax.axis_index("s")
        wid = c * NS + s
        ebase = wid * per_w
        row0 = s * sl

        z16 = jnp.zeros((16,), F32)
        o16 = jnp.ones((16,), F32)
        for j in range(CH // 16):
            ones_r[pl.ds(j * 16, 16)] = o16

        @pl.loop(0, 64)
        def _zero_rows(i):
            for j in range(D // 16):
                zbuf[i, pl.ds(j * 16, 16)] = z16

        @pl.loop(0, sl // 16)
        def _zero_cnt(k):
            cbuf[pl.ds(k * 16, 16)] = z16

        for off, n in _chunks_of(sl, 64):
            pltpu.sync_copy(zbuf.at[pl.ds(0, n)], acc.at[pl.ds(row0 + off, n)])
        pltpu.sync_copy(cbuf.at[pl.ds(0, sl)], cnt.at[pl.ds(row0, sl)])
        plsc.subcore_barrier()

        @pl.loop(0, nch)
        def _edges(j):
            base = ebase + j * CH
            pltpu.sync_copy(src.at[pl.ds(base, CH)], src_idx)
            gd = pltpu.async_copy(table.at[src_idx], rows, gsem)
            pltpu.sync_copy(dst.at[pl.ds(base, CH)], dst_idx)
            pltpu.sync_copy(ones_r, cnt.at[dst_idx], add=True)
            gd.wait()
            pltpu.sync_copy(rows, acc.at[dst_idx], add=True)

        if tail:
            base = ebase + nch * CH
            pltpu.sync_copy(src.at[pl.ds(base, tail)], src_t)
            gd = pltpu.async_copy(table.at[src_t], rows.at[pl.ds(0, tail)], gsem)
            pltpu.sync_copy(dst.at[pl.ds(base, tail)], dst_t)
            pltpu.sync_copy(ones_r.at[pl.ds(0, tail)], cnt.at[dst_t], add=True)
            gd.wait()
            pltpu.sync_copy(rows.at[pl.ds(0, tail)], acc.at[dst_t], add=True)

        plsc.subcore_barrier()

        for off, n in _chunks_of(sl, 64):
            pltpu.sync_copy(acc.at[pl.ds(row0 + off, n)], zbuf.at[pl.ds(0, n)])
            pltpu.sync_copy(zbuf.at[pl.ds(0, n)],
                            sum_out.at[c, pl.ds(row0 + off, n)])
        @pl.when(s == 0)
        def _write_cnt():
            pltpu.sync_copy(cnt, cbuf)
            pltpu.sync_copy(cbuf, cnt_out.at[pl.ds(c * Np, Np)])

    return agg


def _make_tc_dense(Np, act):
    blk = 512
    grid = Np // blk
    dn = (((1,), (1,)), ((), ()))

    def body(p_ref0, p_ref1, cnt_ref, x_ref, wl_ref, wr_ref, b_ref, o_ref):
        i = pl.program_id(0)
        ssum = p_ref0[0] + p_ref1[0]
        cb = cnt_ref[:, pl.ds(i * blk, blk)]
        csum = jnp.maximum(cb[0] + cb[1], 1.0)
        mean = ssum * (1.0 / csum)[:, None]
        h = (lax.dot_general(mean, wl_ref[...], dn, preferred_element_type=F32)
             + lax.dot_general(x_ref[...], wr_ref[...], dn,
                               preferred_element_type=F32)
             + b_ref[...])
        if act == "relu":
            h = jnp.maximum(h, 0.0)
        else:
            m = jnp.max(h, axis=1, keepdims=True)
            e = jnp.exp(h - m)
            h = h - m - jnp.log(jnp.sum(e, axis=1, keepdims=True))
        o_ref[...] = h

    return pl.pallas_call(
        body,
        grid=(grid,),
        in_specs=[
            pl.BlockSpec((1, blk, D), lambda i: (0, i, 0)),
            pl.BlockSpec((1, blk, D), lambda i: (1, i, 0)),
            pl.BlockSpec((NC, Np), lambda i: (0, 0)),
            pl.BlockSpec((blk, D), lambda i: (i, 0)),
            pl.BlockSpec((D, D), lambda i: (0, 0)),
            pl.BlockSpec((D, D), lambda i: (0, 0)),
            pl.BlockSpec((1, D), lambda i: (0, 0)),
        ],
        out_specs=pl.BlockSpec((blk, D), lambda i: (i, 0)),
        out_shape=jax.ShapeDtypeStruct((Np, D), F32),
    )


_agg1 = _make_sc_agg(N0, E1, N1P)
_agg2 = _make_sc_agg(N1P, E2, N2P)
_dense1 = _make_tc_dense(N1P, "relu")
_dense2 = _make_tc_dense(N2P, "logsoftmax")


def kernel(x, edge_index1, edge_index2, W1l, b1l, W1r, W2l, b2l, W2r):
    src1, dst1 = edge_index1[0], edge_index1[1]
    src2, dst2 = edge_index2[0], edge_index2[1]
    b1 = jnp.reshape(b1l, (1, D))
    b2 = jnp.reshape(b2l, (1, D))

    sum1, cnt1 = _agg1(x, src1, dst1)
    h = _dense1(sum1, sum1, cnt1.reshape(NC, N1P), x[:N1P], W1l, W1r, b1)
    sum2, cnt2 = _agg2(h, src2, dst2)
    out = _dense2(sum2, sum2, cnt2.reshape(NC, N2P), h[:N2P], W2l, W2r, b2)
    return out[:N2]
